# baseline (speedup 1.0000x reference)
import functools

import jax
import jax.numpy as jnp
import numpy as np
from jax import lax
from jax.experimental import pallas as pl
from jax.experimental.pallas import tpu as pltpu
from jax.experimental.pallas import tpu_sc as plsc

F32 = jnp.float32
I32 = jnp.int32

NV = 6890
NN = 689
KINF = 3
NEIGH = 18
NC = 2
NS = 16
NW = NC * NS
L = 16

VCHUNK = 224
NVP = VCHUNK * NW
NNP = 704
NPAIR = NN * NEIGH
PCHUNK = 400
NPP = PCHUNK * NW

B_W = NV + 6
B_T = B_W + NV + 6
B_I = B_T + NN + 15
BROW = 21760

SEC_NIDX = 3 * BROW
SEC_RING = SEC_NIDX + NNP
SEC_REP = SEC_RING + NPP
BIG_LEN = SEC_REP + NPP


def _rodrigues_body(rv_ref, out_ref):
    eps = jnp.asarray(1e-8, F32)
    rx = rv_ref[0:1, :]
    ry = rv_ref[1:2, :]
    rz = rv_ref[2:3, :]
    ang = jnp.sqrt((rx + eps) ** 2 + (ry + eps) ** 2 + (rz + eps) ** 2)
    ax = rx / ang
    ay = ry / ang
    az = rz / ang
    c = jnp.cos(ang)
    s = jnp.sin(ang)
    oc = 1.0 - c
    r00 = c + oc * ax * ax
    r01 = oc * ax * ay - s * az
    r02 = oc * ax * az + s * ay
    r10 = oc * ax * ay + s * az
    r11 = c + oc * ay * ay
    r12 = oc * ay * az - s * ax
    r20 = oc * ax * az - s * ay
    r21 = oc * ay * az + s * ax
    r22 = c + oc * az * az
    z = jnp.zeros_like(r00)
    out_ref[...] = jnp.concatenate(
        [r00, r01, r02, r10, r11, r12, r20, r21, r22, z, z, z, z, z, z, z],
        axis=0,
    )


_rodrigues = pl.pallas_call(
    _rodrigues_body,
    out_shape=jax.ShapeDtypeStruct((16, NNP), F32),
)


def _reduce_body(x_ref, o_ref):
    o_ref[0, 0] = jnp.sum(x_ref[...]) / jnp.asarray(float(NN), F32)


_reduce = pl.pallas_call(
    _reduce_body,
    out_shape=jax.ShapeDtypeStruct((1, 1), F32),
    out_specs=pl.BlockSpec(memory_space=pltpu.SMEM),
)


def _sc_body(*refs):
    (big_h, r_h,
     ox_h, oy_h, oz_h, loss_h,
     vxl, vyl, vzl) = refs[:9]
    rl = refs[9:18]
    (txl, tyl, tzl, nidxl,
     nxl, nyl, nzl, pxl, pyl, pzl, bxl, byl, bzl,
     i0l, i1l, i2l, w0l, w1l,
     oxl, oyl, ozl, repl, ringl, accs, sem) = refs[18:]

    wid = lax.axis_index("s") * NC + lax.axis_index("c")
    vbase = pl.multiple_of(wid * VCHUNK, 8)
    pbase = pl.multiple_of(wid * PCHUNK, 8)

    copies = [
        pltpu.make_async_copy(big_h.at[pl.ds(0 * BROW, NVP)], vxl, sem),
        pltpu.make_async_copy(big_h.at[pl.ds(1 * BROW, NVP)], vyl, sem),
        pltpu.make_async_copy(big_h.at[pl.ds(2 * BROW, NVP)], vzl, sem),
        pltpu.make_async_copy(big_h.at[pl.ds(SEC_NIDX, NNP)], nidxl, sem),
    ]
    copies += [pltpu.make_async_copy(r_h.at[j], rl[j], sem)
               for j in range(9)]
    copies += [
        pltpu.make_async_copy(big_h.at[pl.ds(0 * BROW + B_T, NNP)], txl, sem),
        pltpu.make_async_copy(big_h.at[pl.ds(1 * BROW + B_T, NNP)], tyl, sem),
        pltpu.make_async_copy(big_h.at[pl.ds(2 * BROW + B_T, NNP)], tzl, sem),
        pltpu.make_async_copy(
            big_h.at[pl.ds(0 * BROW + B_I + vbase, VCHUNK)], i0l, sem),
        pltpu.make_async_copy(
            big_h.at[pl.ds(1 * BROW + B_I + vbase, VCHUNK)], i1l, sem),
        pltpu.make_async_copy(
            big_h.at[pl.ds(2 * BROW + B_I + vbase, VCHUNK)], i2l, sem),
        pltpu.make_async_copy(
            big_h.at[pl.ds(0 * BROW + B_W + vbase, VCHUNK)], w0l, sem),
        pltpu.make_async_copy(
            big_h.at[pl.ds(1 * BROW + B_W + vbase, VCHUNK)], w1l, sem),
        pltpu.make_async_copy(
            big_h.at[pl.ds(SEC_REP + pbase, PCHUNK)], repl, sem),
        pltpu.make_async_copy(
            big_h.at[pl.ds(SEC_RING + pbase, PCHUNK)], ringl, sem),
    ]
    for c in copies:
        c.start()
    for c in copies:
        c.wait()

    def prep(i, carry):
        s = pl.ds(i * L, L)
        nv = plsc.bitcast(nidxl[s], I32)
        nx = plsc.load_gather(vxl, [nv])
        ny = plsc.load_gather(vyl, [nv])
        nz = plsc.load_gather(vzl, [nv])
        nxl[s] = nx
        nyl[s] = ny
        nzl[s] = nz
        px = nx + txl[s]
        py = ny + tyl[s]
        pz = nz + tzl[s]
        pxl[s] = px
        pyl[s] = py
        pzl[s] = pz
        bxl[s] = px - (rl[0][s] * nx + rl[1][s] * ny + rl[2][s] * nz)
        byl[s] = py - (rl[3][s] * nx + rl[4][s] * ny + rl[5][s] * nz)
        bzl[s] = pz - (rl[6][s] * nx + rl[7][s] * ny + rl[8][s] * nz)
        return carry

    lax.fori_loop(0, NNP // L, prep, 0, unroll=False)

    ils = (i0l, i1l, i2l)

    def warp(i, carry):
        s = pl.ds(i * L, L)
        w0 = w0l[s]
        w1 = w1l[s]
        wks = (w0, w1, 1.0 - w0 - w1)
        zero = jnp.zeros((L,), F32)
        m = [zero] * 9
        cx = zero
        cy = zero
        cz = zero
        for k in range(KINF):
            nk = plsc.bitcast(ils[k][s], I32)
            wk = wks[k]
            for j in range(9):
                m[j] = m[j] + wk * plsc.load_gather(rl[j], [nk])
            cx = cx + wk * plsc.load_gather(bxl, [nk])
            cy = cy + wk * plsc.load_gather(byl, [nk])
            cz = cz + wk * plsc.load_gather(bzl, [nk])
        sv = pl.ds(vbase + i * L, L)
        vx = vxl[sv]
        vy = vyl[sv]
        vz = vzl[sv]
        oxl[s] = m[0] * vx + m[1] * vy + m[2] * vz + cx
        oyl[s] = m[3] * vx + m[4] * vy + m[5] * vz + cy
        ozl[s] = m[6] * vx + m[7] * vy + m[8] * vz + cz
        return carry

    lax.fori_loop(0, VCHUNK // L, warp, 0, unroll=False)

    out_copies = [
        pltpu.make_async_copy(oxl, ox_h.at[pl.ds(vbase, VCHUNK)], sem),
        pltpu.make_async_copy(oyl, oy_h.at[pl.ds(vbase, VCHUNK)], sem),
        pltpu.make_async_copy(ozl, oz_h.at[pl.ds(vbase, VCHUNK)], sem),
    ]
    for c in out_copies:
        c.start()

    def arap(i, acc):
        s = pl.ds(i * L, L)
        bi = plsc.bitcast(repl[s], I32)
        ri = plsc.bitcast(ringl[s], I32)
        bx = plsc.load_gather(bxl, [bi])
        by = plsc.load_gather(byl, [bi])
        bz = plsc.load_gather(bzl, [bi])
        px = plsc.load_gather(pxl, [ri])
        py = plsc.load_gather(pyl, [ri])
        pz = plsc.load_gather(pzl, [ri])
        nx = plsc.load_gather(nxl, [ri])
        ny = plsc.load_gather(nyl, [ri])
        nz = plsc.load_gather(nzl, [ri])
        g = [plsc.load_gather(rl[j], [bi]) for j in range(9)]
        dx = bx - px + (g[0] * nx + g[1] * ny + g[2] * nz)
        dy = by - py + (g[3] * nx + g[4] * ny + g[5] * nz)
        dz = bz - pz + (g[6] * nx + g[7] * ny + g[8] * nz)
        return acc + dx * dx + dy * dy + dz * dz

    acc = lax.fori_loop(0, PCHUNK // L, arap, jnp.zeros((L,), F32),
                        unroll=False)
    accs[...] = acc
    loss_copy = pltpu.make_async_copy(accs, loss_h.at[wid], sem)
    loss_copy.start()
    for c in out_copies:
        c.wait()
    loss_copy.wait()


_sc_main = functools.partial(
    pl.kernel,
    out_type=[
        jax.ShapeDtypeStruct((NVP,), F32),
        jax.ShapeDtypeStruct((NVP,), F32),
        jax.ShapeDtypeStruct((NVP,), F32),
        jax.ShapeDtypeStruct((NW, L), F32),
    ],
    mesh=plsc.VectorSubcoreMesh(
        core_axis_name="c", subcore_axis_name="s",
        num_cores=NC, num_subcores=NS,
    ),
    compiler_params=pltpu.CompilerParams(needs_layout_passes=False),
    scratch_types=(
        [pltpu.VMEM((NVP,), F32)] * 3
        + [pltpu.VMEM((NNP,), F32)] * 9
        + [pltpu.VMEM((NNP,), F32)] * 3
        + [pltpu.VMEM((NNP,), F32)]
        + [pltpu.VMEM((NNP,), F32)] * 9
        + [pltpu.VMEM((VCHUNK,), F32)] * 3
        + [pltpu.VMEM((VCHUNK,), F32)] * 2
        + [pltpu.VMEM((VCHUNK,), F32)] * 3
        + [pltpu.VMEM((PCHUNK,), F32)] * 2
        + [pltpu.VMEM((L,), F32)]
        + [pltpu.SemaphoreType.DMA]
    ),
)(_sc_body)


def _pad1(x, n):
    return jnp.pad(x, (0, n - x.shape[0]))


_Z6 = np.zeros((6, 3), np.float32)
_Z15 = np.zeros((15, 3), np.float32)
_Z15F = np.zeros((15,), np.float32)
_Z398F = np.zeros((398,), np.float32)
_REPF = np.pad(np.repeat(np.arange(NN, dtype=np.int32), NEIGH),
               (0, NPP - NPAIR)).view(np.float32)


def kernel(vertices, opt_d_rotations, opt_d_translations, weights,
           nodes_idx, influence_nodes_idx, one_ring_neigh):
    v = vertices.astype(F32)
    w = weights.astype(F32)
    t = opt_d_translations.reshape(NN, 3).astype(F32)
    iif = lax.bitcast_convert_type(influence_nodes_idx.astype(I32), F32)
    cat = jnp.concatenate([v, _Z6, w, _Z6, t, _Z15, iif], axis=0)
    nidxf = lax.bitcast_convert_type(nodes_idx.astype(I32), F32)
    ringf = lax.bitcast_convert_type(
        one_ring_neigh.astype(I32).reshape(-1), F32)
    big = jnp.concatenate([
        jnp.pad(cat.T, ((0, 0), (0, BROW - cat.shape[0]))).reshape(-1),
        nidxf, _Z15F, ringf, _Z398F, _REPF,
    ])

    rv = opt_d_rotations.reshape(NN, 3).astype(F32)
    rv8 = jnp.zeros((8, NNP), F32)
    rv8 = rv8.at[0, :NN].set(rv[:, 0])
    rv8 = rv8.at[1, :NN].set(rv[:, 1])
    rv8 = rv8.at[2, :NN].set(rv[:, 2])
    rmat = _rodrigues(rv8)

    ox, oy, oz, lossp = _sc_main(big, rmat)

    warped = jnp.stack([ox[:NV], oy[:NV], oz[:NV]], axis=1)[None]
    loss = _reduce(lossp)[0, 0]
    return warped, loss


# speedup vs baseline: 1.1100x; 1.1100x over previous
import functools

import jax
import jax.numpy as jnp
import numpy as np
from jax import lax
from jax.experimental import pallas as pl
from jax.experimental.pallas import tpu as pltpu
from jax.experimental.pallas import tpu_sc as plsc

F32 = jnp.float32
I32 = jnp.int32

NV = 6890
NN = 689
KINF = 3
NEIGH = 18
NC = 2
NS = 16
NW = NC * NS
L = 16

VCHUNK = 224
NVP = VCHUNK * NW
NNP = 704
NPAIR = NN * NEIGH
PCHUNK = 400
NPP = PCHUNK * NW

B_W = NV + 6
B_T = B_W + NV + 6
B_I = B_T + NN + 15
BROW = 21760


def _rodrigues_body(rv_ref, out_ref):
    eps = jnp.asarray(1e-8, F32)
    rx = rv_ref[0:1, :]
    ry = rv_ref[1:2, :]
    rz = rv_ref[2:3, :]
    ang = jnp.sqrt((rx + eps) ** 2 + (ry + eps) ** 2 + (rz + eps) ** 2)
    ax = rx / ang
    ay = ry / ang
    az = rz / ang
    c = jnp.cos(ang)
    s = jnp.sin(ang)
    oc = 1.0 - c
    r00 = c + oc * ax * ax
    r01 = oc * ax * ay - s * az
    r02 = oc * ax * az + s * ay
    r10 = oc * ax * ay + s * az
    r11 = c + oc * ay * ay
    r12 = oc * ay * az - s * ax
    r20 = oc * ax * az - s * ay
    r21 = oc * ay * az + s * ax
    r22 = c + oc * az * az
    z = jnp.zeros_like(r00)
    out_ref[...] = jnp.concatenate(
        [r00, r01, r02, r10, r11, r12, r20, r21, r22, z, z, z, z, z, z, z],
        axis=0,
    )


_rodrigues = pl.pallas_call(
    _rodrigues_body,
    out_shape=jax.ShapeDtypeStruct((16, NNP), F32),
)


def _reduce_body(x_ref, o_ref):
    o_ref[0, 0] = jnp.sum(x_ref[...]) / jnp.asarray(float(NN), F32)


_reduce = pl.pallas_call(
    _reduce_body,
    out_shape=jax.ShapeDtypeStruct((1, 1), F32),
    out_specs=pl.BlockSpec(memory_space=pltpu.SMEM),
)


def _sc_body(*refs):
    (big_h, r_h, nidx_h, rep_h, ring_h,
     ox_h, oy_h, oz_h, loss_h,
     vxl, vyl, vzl) = refs[:12]
    rl = refs[12:21]
    (txl, tyl, tzl, nidxl,
     nxl, nyl, nzl, pxl, pyl, pzl, bxl, byl, bzl,
     i0l, i1l, i2l, w0l, w1l,
     oxl, oyl, ozl, repl, ringl, accs, sem) = refs[21:]

    wid = lax.axis_index("s") * NC + lax.axis_index("c")
    vbase = pl.multiple_of(wid * VCHUNK, 8)
    pbase = pl.multiple_of(wid * PCHUNK, 8)

    copies = [
        pltpu.make_async_copy(big_h.at[pl.ds(0 * BROW, NVP)], vxl, sem),
        pltpu.make_async_copy(big_h.at[pl.ds(1 * BROW, NVP)], vyl, sem),
        pltpu.make_async_copy(big_h.at[pl.ds(2 * BROW, NVP)], vzl, sem),
        pltpu.make_async_copy(nidx_h, nidxl, sem),
    ]
    copies += [pltpu.make_async_copy(r_h.at[j], rl[j], sem)
               for j in range(9)]
    copies += [
        pltpu.make_async_copy(big_h.at[pl.ds(0 * BROW + B_T, NNP)], txl, sem),
        pltpu.make_async_copy(big_h.at[pl.ds(1 * BROW + B_T, NNP)], tyl, sem),
        pltpu.make_async_copy(big_h.at[pl.ds(2 * BROW + B_T, NNP)], tzl, sem),
        pltpu.make_async_copy(
            big_h.at[pl.ds(0 * BROW + B_I + vbase, VCHUNK)], i0l, sem),
        pltpu.make_async_copy(
            big_h.at[pl.ds(1 * BROW + B_I + vbase, VCHUNK)], i1l, sem),
        pltpu.make_async_copy(
            big_h.at[pl.ds(2 * BROW + B_I + vbase, VCHUNK)], i2l, sem),
        pltpu.make_async_copy(
            big_h.at[pl.ds(0 * BROW + B_W + vbase, VCHUNK)], w0l, sem),
        pltpu.make_async_copy(
            big_h.at[pl.ds(1 * BROW + B_W + vbase, VCHUNK)], w1l, sem),
        pltpu.make_async_copy(rep_h.at[pl.ds(pbase, PCHUNK)], repl, sem),
        pltpu.make_async_copy(ring_h.at[pl.ds(pbase, PCHUNK)], ringl, sem),
    ]
    for c in copies:
        c.start()
    for c in copies:
        c.wait()

    def prep(i, carry):
        s = pl.ds(i * L, L)
        nv = nidxl[s]
        nx = plsc.load_gather(vxl, [nv])
        ny = plsc.load_gather(vyl, [nv])
        nz = plsc.load_gather(vzl, [nv])
        nxl[s] = nx
        nyl[s] = ny
        nzl[s] = nz
        px = nx + txl[s]
        py = ny + tyl[s]
        pz = nz + tzl[s]
        pxl[s] = px
        pyl[s] = py
        pzl[s] = pz
        bxl[s] = px - (rl[0][s] * nx + rl[1][s] * ny + rl[2][s] * nz)
        byl[s] = py - (rl[3][s] * nx + rl[4][s] * ny + rl[5][s] * nz)
        bzl[s] = pz - (rl[6][s] * nx + rl[7][s] * ny + rl[8][s] * nz)
        return carry

    lax.fori_loop(0, NNP // L, prep, 0, unroll=False)

    ils = (i0l, i1l, i2l)

    def warp(i, carry):
        s = pl.ds(i * L, L)
        w0 = w0l[s]
        w1 = w1l[s]
        wks = (w0, w1, 1.0 - w0 - w1)
        zero = jnp.zeros((L,), F32)
        m = [zero] * 9
        cx = zero
        cy = zero
        cz = zero
        for k in range(KINF):
            nk = plsc.bitcast(ils[k][s], I32)
            wk = wks[k]
            for j in range(9):
                m[j] = m[j] + wk * plsc.load_gather(rl[j], [nk])
            cx = cx + wk * plsc.load_gather(bxl, [nk])
            cy = cy + wk * plsc.load_gather(byl, [nk])
            cz = cz + wk * plsc.load_gather(bzl, [nk])
        sv = pl.ds(vbase + i * L, L)
        vx = vxl[sv]
        vy = vyl[sv]
        vz = vzl[sv]
        oxl[s] = m[0] * vx + m[1] * vy + m[2] * vz + cx
        oyl[s] = m[3] * vx + m[4] * vy + m[5] * vz + cy
        ozl[s] = m[6] * vx + m[7] * vy + m[8] * vz + cz
        return carry

    lax.fori_loop(0, VCHUNK // L, warp, 0, unroll=False)

    out_copies = [
        pltpu.make_async_copy(oxl, ox_h.at[pl.ds(vbase, VCHUNK)], sem),
        pltpu.make_async_copy(oyl, oy_h.at[pl.ds(vbase, VCHUNK)], sem),
        pltpu.make_async_copy(ozl, oz_h.at[pl.ds(vbase, VCHUNK)], sem),
    ]
    for c in out_copies:
        c.start()

    def arap(i, acc):
        s = pl.ds(i * L, L)
        bi = repl[s]
        ri = ringl[s]
        bx = plsc.load_gather(bxl, [bi])
        by = plsc.load_gather(byl, [bi])
        bz = plsc.load_gather(bzl, [bi])
        px = plsc.load_gather(pxl, [ri])
        py = plsc.load_gather(pyl, [ri])
        pz = plsc.load_gather(pzl, [ri])
        nx = plsc.load_gather(nxl, [ri])
        ny = plsc.load_gather(nyl, [ri])
        nz = plsc.load_gather(nzl, [ri])
        g = [plsc.load_gather(rl[j], [bi]) for j in range(9)]
        dx = bx - px + (g[0] * nx + g[1] * ny + g[2] * nz)
        dy = by - py + (g[3] * nx + g[4] * ny + g[5] * nz)
        dz = bz - pz + (g[6] * nx + g[7] * ny + g[8] * nz)
        return acc + dx * dx + dy * dy + dz * dz

    acc = lax.fori_loop(0, PCHUNK // L, arap, jnp.zeros((L,), F32),
                        unroll=False)
    accs[...] = acc
    loss_copy = pltpu.make_async_copy(accs, loss_h.at[wid], sem)
    loss_copy.start()
    for c in out_copies:
        c.wait()
    loss_copy.wait()


_sc_main = functools.partial(
    pl.kernel,
    out_type=[
        jax.ShapeDtypeStruct((NVP,), F32),
        jax.ShapeDtypeStruct((NVP,), F32),
        jax.ShapeDtypeStruct((NVP,), F32),
        jax.ShapeDtypeStruct((NW, L), F32),
    ],
    mesh=plsc.VectorSubcoreMesh(
        core_axis_name="c", subcore_axis_name="s",
        num_cores=NC, num_subcores=NS,
    ),
    compiler_params=pltpu.CompilerParams(needs_layout_passes=False),
    scratch_types=(
        [pltpu.VMEM((NVP,), F32)] * 3
        + [pltpu.VMEM((NNP,), F32)] * 9
        + [pltpu.VMEM((NNP,), F32)] * 3
        + [pltpu.VMEM((NNP,), I32)]
        + [pltpu.VMEM((NNP,), F32)] * 9
        + [pltpu.VMEM((VCHUNK,), F32)] * 3
        + [pltpu.VMEM((VCHUNK,), F32)] * 2
        + [pltpu.VMEM((VCHUNK,), F32)] * 3
        + [pltpu.VMEM((PCHUNK,), I32)] * 2
        + [pltpu.VMEM((L,), F32)]
        + [pltpu.SemaphoreType.DMA]
    ),
)(_sc_body)


def _pad1(x, n):
    return jnp.pad(x, (0, n - x.shape[0]))


_Z6 = np.zeros((6, 3), np.float32)
_Z15 = np.zeros((15, 3), np.float32)


def kernel(vertices, opt_d_rotations, opt_d_translations, weights,
           nodes_idx, influence_nodes_idx, one_ring_neigh):
    v = vertices.astype(F32)
    w = weights.astype(F32)
    t = opt_d_translations.reshape(NN, 3).astype(F32)
    iif = lax.bitcast_convert_type(influence_nodes_idx.astype(I32), F32)
    cat = jnp.concatenate([v, _Z6, w, _Z6, t, _Z15, iif], axis=0)
    big = jnp.pad(cat.T, ((0, 0), (0, BROW - cat.shape[0]))).reshape(-1)

    rv = opt_d_rotations.reshape(NN, 3).astype(F32)
    rv8 = jnp.pad(rv.T, ((0, 5), (0, NNP - NN)))
    rmat = _rodrigues(rv8)

    nidx = _pad1(nodes_idx.astype(I32), NNP)
    rep = _pad1(jnp.repeat(jnp.arange(NN, dtype=I32), NEIGH), NPP)
    ring = _pad1(one_ring_neigh.astype(I32).reshape(-1), NPP)

    ox, oy, oz, lossp = _sc_main(big, rmat, nidx, rep, ring)

    warped = jnp.stack([ox[:NV], oy[:NV], oz[:NV]], axis=1)[None]
    loss = _reduce(lossp)[0, 0]
    return warped, loss


# speedup vs baseline: 1.1162x; 1.0055x over previous
import functools

import jax
import jax.numpy as jnp
import numpy as np
from jax import lax
from jax.experimental import pallas as pl
from jax.experimental.pallas import tpu as pltpu
from jax.experimental.pallas import tpu_sc as plsc

F32 = jnp.float32
I32 = jnp.int32

NV = 6890
NN = 689
KINF = 3
NEIGH = 18
NC = 2
NS = 16
NW = NC * NS
L = 16

VCHUNK = 224
NVP = VCHUNK * NW
NNP = 704
NPAIR = NN * NEIGH
PCHUNK = 400
NPP = PCHUNK * NW

B_W = NV + 6
B_T = B_W + NV + 6
B_I = B_T + NN + 15
BROW = 21760


def _rodrigues_body(rv_ref, out_ref):
    eps = jnp.asarray(1e-8, F32)
    rx = rv_ref[0:1, :]
    ry = rv_ref[1:2, :]
    rz = rv_ref[2:3, :]
    ang = jnp.sqrt((rx + eps) ** 2 + (ry + eps) ** 2 + (rz + eps) ** 2)
    ax = rx / ang
    ay = ry / ang
    az = rz / ang
    c = jnp.cos(ang)
    s = jnp.sin(ang)
    oc = 1.0 - c
    r00 = c + oc * ax * ax
    r01 = oc * ax * ay - s * az
    r02 = oc * ax * az + s * ay
    r10 = oc * ax * ay + s * az
    r11 = c + oc * ay * ay
    r12 = oc * ay * az - s * ax
    r20 = oc * ax * az - s * ay
    r21 = oc * ay * az + s * ax
    r22 = c + oc * az * az
    z = jnp.zeros_like(r00)
    out_ref[...] = jnp.concatenate(
        [r00, r01, r02, r10, r11, r12, r20, r21, r22, z, z, z, z, z, z, z],
        axis=0,
    )


_rodrigues = pl.pallas_call(
    _rodrigues_body,
    out_shape=jax.ShapeDtypeStruct((16, NNP), F32),
)


def _reduce_body(x_ref, o_ref):
    o_ref[0, 0] = jnp.sum(x_ref[...]) / jnp.asarray(float(NN), F32)


_reduce = pl.pallas_call(
    _reduce_body,
    out_shape=jax.ShapeDtypeStruct((1, 1), F32),
    out_specs=pl.BlockSpec(memory_space=pltpu.SMEM),
)


def _sc_body(*refs):
    (big_h, r_h, nidx_h, rep_h, ring_h,
     ox_h, oy_h, oz_h, loss_h,
     vxl, vyl, vzl) = refs[:12]
    rl = refs[12:21]
    (txl, tyl, tzl, nidxl,
     nxl, nyl, nzl, pxl, pyl, pzl, bxl, byl, bzl,
     i0l, i1l, i2l, w0l, w1l,
     oxl, oyl, ozl, repl, ringl, accs, sem) = refs[21:]

    wid = lax.axis_index("s") * NC + lax.axis_index("c")
    vbase = pl.multiple_of(wid * VCHUNK, 8)
    pbase = pl.multiple_of(wid * PCHUNK, 8)

    copies = [
        pltpu.make_async_copy(big_h.at[pl.ds(0 * BROW, NVP)], vxl, sem),
        pltpu.make_async_copy(big_h.at[pl.ds(1 * BROW, NVP)], vyl, sem),
        pltpu.make_async_copy(big_h.at[pl.ds(2 * BROW, NVP)], vzl, sem),
        pltpu.make_async_copy(nidx_h, nidxl, sem),
    ]
    copies += [pltpu.make_async_copy(r_h.at[j], rl[j], sem)
               for j in range(9)]
    copies += [
        pltpu.make_async_copy(big_h.at[pl.ds(0 * BROW + B_T, NNP)], txl, sem),
        pltpu.make_async_copy(big_h.at[pl.ds(1 * BROW + B_T, NNP)], tyl, sem),
        pltpu.make_async_copy(big_h.at[pl.ds(2 * BROW + B_T, NNP)], tzl, sem),
        pltpu.make_async_copy(
            big_h.at[pl.ds(0 * BROW + B_I + vbase, VCHUNK)], i0l, sem),
        pltpu.make_async_copy(
            big_h.at[pl.ds(1 * BROW + B_I + vbase, VCHUNK)], i1l, sem),
        pltpu.make_async_copy(
            big_h.at[pl.ds(2 * BROW + B_I + vbase, VCHUNK)], i2l, sem),
        pltpu.make_async_copy(
            big_h.at[pl.ds(0 * BROW + B_W + vbase, VCHUNK)], w0l, sem),
        pltpu.make_async_copy(
            big_h.at[pl.ds(1 * BROW + B_W + vbase, VCHUNK)], w1l, sem),
        pltpu.make_async_copy(rep_h.at[pl.ds(pbase, PCHUNK)], repl, sem),
        pltpu.make_async_copy(ring_h.at[pl.ds(pbase, PCHUNK)], ringl, sem),
    ]
    for c in copies:
        c.start()
    for c in copies:
        c.wait()

    def prep(i, carry):
        s = pl.ds(i * L, L)
        nv = nidxl[s]
        nx = plsc.load_gather(vxl, [nv])
        ny = plsc.load_gather(vyl, [nv])
        nz = plsc.load_gather(vzl, [nv])
        nxl[s] = nx
        nyl[s] = ny
        nzl[s] = nz
        px = nx + txl[s]
        py = ny + tyl[s]
        pz = nz + tzl[s]
        pxl[s] = px
        pyl[s] = py
        pzl[s] = pz
        bxl[s] = px - (rl[0][s] * nx + rl[1][s] * ny + rl[2][s] * nz)
        byl[s] = py - (rl[3][s] * nx + rl[4][s] * ny + rl[5][s] * nz)
        bzl[s] = pz - (rl[6][s] * nx + rl[7][s] * ny + rl[8][s] * nz)
        return carry

    lax.fori_loop(0, NNP // L, prep, 0, unroll=False)

    ils = (i0l, i1l, i2l)

    def warp(i, carry):
        s = pl.ds(i * L, L)
        w0 = w0l[s]
        w1 = w1l[s]
        wks = (w0, w1, 1.0 - w0 - w1)
        zero = jnp.zeros((L,), F32)
        m = [zero] * 9
        cx = zero
        cy = zero
        cz = zero
        for k in range(KINF):
            nk = plsc.bitcast(ils[k][s], I32)
            wk = wks[k]
            for j in range(9):
                m[j] = m[j] + wk * plsc.load_gather(rl[j], [nk])
            cx = cx + wk * plsc.load_gather(bxl, [nk])
            cy = cy + wk * plsc.load_gather(byl, [nk])
            cz = cz + wk * plsc.load_gather(bzl, [nk])
        sv = pl.ds(vbase + i * L, L)
        vx = vxl[sv]
        vy = vyl[sv]
        vz = vzl[sv]
        oxl[s] = m[0] * vx + m[1] * vy + m[2] * vz + cx
        oyl[s] = m[3] * vx + m[4] * vy + m[5] * vz + cy
        ozl[s] = m[6] * vx + m[7] * vy + m[8] * vz + cz
        return carry

    lax.fori_loop(0, VCHUNK // L, warp, 0, unroll=2)

    out_copies = [
        pltpu.make_async_copy(oxl, ox_h.at[pl.ds(vbase, VCHUNK)], sem),
        pltpu.make_async_copy(oyl, oy_h.at[pl.ds(vbase, VCHUNK)], sem),
        pltpu.make_async_copy(ozl, oz_h.at[pl.ds(vbase, VCHUNK)], sem),
    ]
    for c in out_copies:
        c.start()

    def arap(i, acc):
        s = pl.ds(i * L, L)
        bi = repl[s]
        ri = ringl[s]
        bx = plsc.load_gather(bxl, [bi])
        by = plsc.load_gather(byl, [bi])
        bz = plsc.load_gather(bzl, [bi])
        px = plsc.load_gather(pxl, [ri])
        py = plsc.load_gather(pyl, [ri])
        pz = plsc.load_gather(pzl, [ri])
        nx = plsc.load_gather(nxl, [ri])
        ny = plsc.load_gather(nyl, [ri])
        nz = plsc.load_gather(nzl, [ri])
        g = [plsc.load_gather(rl[j], [bi]) for j in range(9)]
        dx = bx - px + (g[0] * nx + g[1] * ny + g[2] * nz)
        dy = by - py + (g[3] * nx + g[4] * ny + g[5] * nz)
        dz = bz - pz + (g[6] * nx + g[7] * ny + g[8] * nz)
        return acc + dx * dx + dy * dy + dz * dz

    acc = lax.fori_loop(0, PCHUNK // L, arap, jnp.zeros((L,), F32),
                        unroll=2)
    accs[...] = acc
    loss_copy = pltpu.make_async_copy(accs, loss_h.at[wid], sem)
    loss_copy.start()
    for c in out_copies:
        c.wait()
    loss_copy.wait()


_sc_main = functools.partial(
    pl.kernel,
    out_type=[
        jax.ShapeDtypeStruct((NVP,), F32),
        jax.ShapeDtypeStruct((NVP,), F32),
        jax.ShapeDtypeStruct((NVP,), F32),
        jax.ShapeDtypeStruct((NW, L), F32),
    ],
    mesh=plsc.VectorSubcoreMesh(
        core_axis_name="c", subcore_axis_name="s",
        num_cores=NC, num_subcores=NS,
    ),
    compiler_params=pltpu.CompilerParams(needs_layout_passes=False),
    scratch_types=(
        [pltpu.VMEM((NVP,), F32)] * 3
        + [pltpu.VMEM((NNP,), F32)] * 9
        + [pltpu.VMEM((NNP,), F32)] * 3
        + [pltpu.VMEM((NNP,), I32)]
        + [pltpu.VMEM((NNP,), F32)] * 9
        + [pltpu.VMEM((VCHUNK,), F32)] * 3
        + [pltpu.VMEM((VCHUNK,), F32)] * 2
        + [pltpu.VMEM((VCHUNK,), F32)] * 3
        + [pltpu.VMEM((PCHUNK,), I32)] * 2
        + [pltpu.VMEM((L,), F32)]
        + [pltpu.SemaphoreType.DMA]
    ),
)(_sc_body)


def _pad1(x, n):
    return jnp.pad(x, (0, n - x.shape[0]))


_Z6 = np.zeros((6, 3), np.float32)
_Z15 = np.zeros((15, 3), np.float32)


def kernel(vertices, opt_d_rotations, opt_d_translations, weights,
           nodes_idx, influence_nodes_idx, one_ring_neigh):
    v = vertices.astype(F32)
    w = weights.astype(F32)
    t = opt_d_translations.reshape(NN, 3).astype(F32)
    iif = lax.bitcast_convert_type(influence_nodes_idx.astype(I32), F32)
    cat = jnp.concatenate([v, _Z6, w, _Z6, t, _Z15, iif], axis=0)
    big = jnp.pad(cat.T, ((0, 0), (0, BROW - cat.shape[0]))).reshape(-1)

    rv = opt_d_rotations.reshape(NN, 3).astype(F32)
    rv8 = jnp.zeros((8, NNP), F32)
    rv8 = rv8.at[0, :NN].set(rv[:, 0])
    rv8 = rv8.at[1, :NN].set(rv[:, 1])
    rv8 = rv8.at[2, :NN].set(rv[:, 2])
    rmat = _rodrigues(rv8)

    nidx = _pad1(nodes_idx.astype(I32), NNP)
    rep = _pad1(jnp.repeat(jnp.arange(NN, dtype=I32), NEIGH), NPP)
    ring = _pad1(one_ring_neigh.astype(I32).reshape(-1), NPP)

    ox, oy, oz, lossp = _sc_main(big, rmat, nidx, rep, ring)

    warped = jnp.stack([ox[:NV], oy[:NV], oz[:NV]], axis=1)[None]
    loss = _reduce(lossp)[0, 0]
    return warped, loss


# speedup vs baseline: 1.1252x; 1.0081x over previous
import functools

import jax
import jax.numpy as jnp
import numpy as np
from jax import lax
from jax.experimental import pallas as pl
from jax.experimental.pallas import tpu as pltpu
from jax.experimental.pallas import tpu_sc as plsc

F32 = jnp.float32
I32 = jnp.int32

NV = 6890
NN = 689
KINF = 3
NEIGH = 18
NC = 2
NS = 16
NW = NC * NS
L = 16

VCHUNK = 224
NVP = VCHUNK * NW
NNP = 704
NPAIR = NN * NEIGH
PCHUNK = 400
NPP = PCHUNK * NW

B_W = NV + 6
B_T = B_W + NV + 6
B_I = B_T + NN + 15
BROW = 21760


def _rodrigues_body(rv_ref, out_ref):
    eps = jnp.asarray(1e-8, F32)
    rx = rv_ref[0:1, :]
    ry = rv_ref[1:2, :]
    rz = rv_ref[2:3, :]
    ang = jnp.sqrt((rx + eps) ** 2 + (ry + eps) ** 2 + (rz + eps) ** 2)
    ax = rx / ang
    ay = ry / ang
    az = rz / ang
    c = jnp.cos(ang)
    s = jnp.sin(ang)
    oc = 1.0 - c
    r00 = c + oc * ax * ax
    r01 = oc * ax * ay - s * az
    r02 = oc * ax * az + s * ay
    r10 = oc * ax * ay + s * az
    r11 = c + oc * ay * ay
    r12 = oc * ay * az - s * ax
    r20 = oc * ax * az - s * ay
    r21 = oc * ay * az + s * ax
    r22 = c + oc * az * az
    z = jnp.zeros_like(r00)
    out_ref[...] = jnp.concatenate(
        [r00, r01, r02, r10, r11, r12, r20, r21, r22, z, z, z, z, z, z, z],
        axis=0,
    )


_rodrigues = pl.pallas_call(
    _rodrigues_body,
    out_shape=jax.ShapeDtypeStruct((16, NNP), F32),
)


def _reduce_body(x_ref, o_ref):
    o_ref[0, 0] = jnp.sum(x_ref[...]) / jnp.asarray(float(NN), F32)


_reduce = pl.pallas_call(
    _reduce_body,
    out_shape=jax.ShapeDtypeStruct((1, 1), F32),
    out_specs=pl.BlockSpec(memory_space=pltpu.SMEM),
)


def _sc_body(*refs):
    (big_h, r_h, nidx_h, rep_h, ring_h,
     ox_h, oy_h, oz_h, loss_h,
     vxl, vyl, vzl) = refs[:12]
    rl = refs[12:21]
    (txl, tyl, tzl, nidxl,
     nxl, nyl, nzl, pxl, pyl, pzl, bxl, byl, bzl,
     i0l, i1l, i2l, w0l, w1l,
     oxl, oyl, ozl, repl, ringl, accs, sem) = refs[21:]

    wid = lax.axis_index("s") * NC + lax.axis_index("c")
    vbase = pl.multiple_of(wid * VCHUNK, 8)
    pbase = pl.multiple_of(wid * PCHUNK, 8)

    copies = [
        pltpu.make_async_copy(big_h.at[pl.ds(0 * BROW, NVP)], vxl, sem),
        pltpu.make_async_copy(big_h.at[pl.ds(1 * BROW, NVP)], vyl, sem),
        pltpu.make_async_copy(big_h.at[pl.ds(2 * BROW, NVP)], vzl, sem),
        pltpu.make_async_copy(nidx_h, nidxl, sem),
    ]
    copies += [pltpu.make_async_copy(r_h.at[j], rl[j], sem)
               for j in range(9)]
    copies += [
        pltpu.make_async_copy(big_h.at[pl.ds(0 * BROW + B_T, NNP)], txl, sem),
        pltpu.make_async_copy(big_h.at[pl.ds(1 * BROW + B_T, NNP)], tyl, sem),
        pltpu.make_async_copy(big_h.at[pl.ds(2 * BROW + B_T, NNP)], tzl, sem),
        pltpu.make_async_copy(
            big_h.at[pl.ds(0 * BROW + B_I + vbase, VCHUNK)], i0l, sem),
        pltpu.make_async_copy(
            big_h.at[pl.ds(1 * BROW + B_I + vbase, VCHUNK)], i1l, sem),
        pltpu.make_async_copy(
            big_h.at[pl.ds(2 * BROW + B_I + vbase, VCHUNK)], i2l, sem),
        pltpu.make_async_copy(
            big_h.at[pl.ds(0 * BROW + B_W + vbase, VCHUNK)], w0l, sem),
        pltpu.make_async_copy(
            big_h.at[pl.ds(1 * BROW + B_W + vbase, VCHUNK)], w1l, sem),
        pltpu.make_async_copy(rep_h.at[pl.ds(pbase, PCHUNK)], repl, sem),
        pltpu.make_async_copy(ring_h.at[pl.ds(pbase, PCHUNK)], ringl, sem),
    ]
    for c in copies:
        c.start()
    for c in copies:
        c.wait()

    def prep(i, carry):
        s = pl.ds(i * L, L)
        nv = nidxl[s]
        nx = plsc.load_gather(vxl, [nv])
        ny = plsc.load_gather(vyl, [nv])
        nz = plsc.load_gather(vzl, [nv])
        nxl[s] = nx
        nyl[s] = ny
        nzl[s] = nz
        px = nx + txl[s]
        py = ny + tyl[s]
        pz = nz + tzl[s]
        pxl[s] = px
        pyl[s] = py
        pzl[s] = pz
        bxl[s] = px - (rl[0][s] * nx + rl[1][s] * ny + rl[2][s] * nz)
        byl[s] = py - (rl[3][s] * nx + rl[4][s] * ny + rl[5][s] * nz)
        bzl[s] = pz - (rl[6][s] * nx + rl[7][s] * ny + rl[8][s] * nz)
        return carry

    lax.fori_loop(0, NNP // L, prep, 0, unroll=False)

    ils = (i0l, i1l, i2l)

    def warp(i, carry):
        s = pl.ds(i * L, L)
        w0 = w0l[s]
        w1 = w1l[s]
        wks = (w0, w1, 1.0 - w0 - w1)
        zero = jnp.zeros((L,), F32)
        m = [zero] * 9
        cx = zero
        cy = zero
        cz = zero
        for k in range(KINF):
            nk = plsc.bitcast(ils[k][s], I32)
            wk = wks[k]
            for j in range(9):
                m[j] = m[j] + wk * plsc.load_gather(rl[j], [nk])
            cx = cx + wk * plsc.load_gather(bxl, [nk])
            cy = cy + wk * plsc.load_gather(byl, [nk])
            cz = cz + wk * plsc.load_gather(bzl, [nk])
        sv = pl.ds(vbase + i * L, L)
        vx = vxl[sv]
        vy = vyl[sv]
        vz = vzl[sv]
        oxl[s] = m[0] * vx + m[1] * vy + m[2] * vz + cx
        oyl[s] = m[3] * vx + m[4] * vy + m[5] * vz + cy
        ozl[s] = m[6] * vx + m[7] * vy + m[8] * vz + cz
        return carry

    lax.fori_loop(0, VCHUNK // L, warp, 0, unroll=False)

    out_copies = [
        pltpu.make_async_copy(oxl, ox_h.at[pl.ds(vbase, VCHUNK)], sem),
        pltpu.make_async_copy(oyl, oy_h.at[pl.ds(vbase, VCHUNK)], sem),
        pltpu.make_async_copy(ozl, oz_h.at[pl.ds(vbase, VCHUNK)], sem),
    ]
    for c in out_copies:
        c.start()

    def arap(i, acc):
        s = pl.ds(i * L, L)
        bi = repl[s]
        ri = ringl[s]
        bx = plsc.load_gather(bxl, [bi])
        by = plsc.load_gather(byl, [bi])
        bz = plsc.load_gather(bzl, [bi])
        px = plsc.load_gather(pxl, [ri])
        py = plsc.load_gather(pyl, [ri])
        pz = plsc.load_gather(pzl, [ri])
        nx = plsc.load_gather(nxl, [ri])
        ny = plsc.load_gather(nyl, [ri])
        nz = plsc.load_gather(nzl, [ri])
        g = [plsc.load_gather(rl[j], [bi]) for j in range(9)]
        dx = bx - px + (g[0] * nx + g[1] * ny + g[2] * nz)
        dy = by - py + (g[3] * nx + g[4] * ny + g[5] * nz)
        dz = bz - pz + (g[6] * nx + g[7] * ny + g[8] * nz)
        return acc + dx * dx + dy * dy + dz * dz

    acc = lax.fori_loop(0, PCHUNK // L, arap, jnp.zeros((L,), F32),
                        unroll=False)
    accs[...] = acc
    loss_copy = pltpu.make_async_copy(accs, loss_h.at[wid], sem)
    loss_copy.start()
    for c in out_copies:
        c.wait()
    loss_copy.wait()


_sc_main = functools.partial(
    pl.kernel,
    out_type=[
        jax.ShapeDtypeStruct((NVP,), F32),
        jax.ShapeDtypeStruct((NVP,), F32),
        jax.ShapeDtypeStruct((NVP,), F32),
        jax.ShapeDtypeStruct((NW, L), F32),
    ],
    mesh=plsc.VectorSubcoreMesh(
        core_axis_name="c", subcore_axis_name="s",
        num_cores=NC, num_subcores=NS,
    ),
    compiler_params=pltpu.CompilerParams(needs_layout_passes=False),
    scratch_types=(
        [pltpu.VMEM((NVP,), F32)] * 3
        + [pltpu.VMEM((NNP,), F32)] * 9
        + [pltpu.VMEM((NNP,), F32)] * 3
        + [pltpu.VMEM((NNP,), I32)]
        + [pltpu.VMEM((NNP,), F32)] * 9
        + [pltpu.VMEM((VCHUNK,), F32)] * 3
        + [pltpu.VMEM((VCHUNK,), F32)] * 2
        + [pltpu.VMEM((VCHUNK,), F32)] * 3
        + [pltpu.VMEM((PCHUNK,), I32)] * 2
        + [pltpu.VMEM((L,), F32)]
        + [pltpu.SemaphoreType.DMA]
    ),
)(_sc_body)


def _pad1(x, n):
    return jnp.pad(x, (0, n - x.shape[0]))


_Z6 = np.zeros((6, 3), np.float32)
_Z15 = np.zeros((15, 3), np.float32)


def kernel(vertices, opt_d_rotations, opt_d_translations, weights,
           nodes_idx, influence_nodes_idx, one_ring_neigh):
    v = vertices.astype(F32)
    w = weights.astype(F32)
    t = opt_d_translations.reshape(NN, 3).astype(F32)
    iif = lax.bitcast_convert_type(influence_nodes_idx.astype(I32), F32)
    cat = jnp.concatenate([v, _Z6, w, _Z6, t, _Z15, iif], axis=0)
    big = jnp.pad(cat.T, ((0, 0), (0, BROW - cat.shape[0]))).reshape(-1)

    rv = opt_d_rotations.reshape(NN, 3).astype(F32)
    rv8 = jnp.zeros((8, NNP), F32)
    rv8 = rv8.at[0, :NN].set(rv[:, 0])
    rv8 = rv8.at[1, :NN].set(rv[:, 1])
    rv8 = rv8.at[2, :NN].set(rv[:, 2])
    rmat = _rodrigues(rv8)

    nidx = _pad1(nodes_idx.astype(I32), NNP)
    rep = _pad1(jnp.repeat(jnp.arange(NN, dtype=I32), NEIGH), NPP)
    ring = _pad1(one_ring_neigh.astype(I32).reshape(-1), NPP)

    ox, oy, oz, lossp = _sc_main(big, rmat, nidx, rep, ring)

    warped = jnp.stack([ox[:NV], oy[:NV], oz[:NV]], axis=1)[None]
    loss = _reduce(lossp)[0, 0]
    return warped, loss
